# Initial kernel scaffold; baseline (speedup 1.0000x reference)
import jax
import jax.numpy as jnp
from jax.experimental import pallas as pl


def kernel(edge_feat, dist, path, emb_weight):
    raise NotImplementedError("write your pallas kernel here")


# speedup vs baseline: 11.7037x; 11.7037x over previous
import functools

import jax
import jax.numpy as jnp
from jax import lax
from jax.experimental import pallas as pl
from jax.experimental.pallas import tpu as pltpu
from jax.experimental.pallas import tpu_sc as plsc

MAX_LEN = 2
NUM_HEADS = 8
FEAT_DIM = 128
N_GRAPH = 16
MAX_NODES = 128
EDGES_PER_GRAPH = 2048

_NC = 2
_NS = 16
_NW = _NC * _NS
_PAIRS = MAX_NODES * MAX_NODES
_HALF = _PAIRS // 2
_PROJ_W = MAX_LEN * NUM_HEADS
_GSLICE = EDGES_PER_GRAPH * _PROJ_W
_OUT_W = _HALF * NUM_HEADS


def _proj_matmul(edge_feat, emb_weight):
    blk = 4096

    def body(x_ref, w_ref, o_ref):
        o_ref[...] = lax.dot_general(
            x_ref[...], w_ref[...],
            (((1,), (1,)), ((), ())),
            preferred_element_type=jnp.float32,
        )

    return pl.pallas_call(
        body,
        grid=(N_GRAPH * EDGES_PER_GRAPH // blk,),
        in_specs=[
            pl.BlockSpec((blk, FEAT_DIM), lambda i: (i, 0)),
            pl.BlockSpec((_PROJ_W, FEAT_DIM), lambda i: (0, 0)),
        ],
        out_specs=pl.BlockSpec((blk, _PROJ_W), lambda i: (i, 0)),
        out_shape=jax.ShapeDtypeStruct((N_GRAPH * EDGES_PER_GRAPH, _PROJ_W),
                                       jnp.float32),
    )(edge_feat, emb_weight)


def _sc_combine(proj_flat, p0, p1, dist_flat):
    mesh = plsc.VectorSubcoreMesh(
        core_axis_name="c", subcore_axis_name="s",
        num_cores=_NC, num_subcores=_NS)

    @functools.partial(
        pl.kernel,
        out_type=jax.ShapeDtypeStruct((N_GRAPH, _PAIRS * NUM_HEADS),
                                      jnp.float32),
        mesh=mesh,
        compiler_params=pltpu.CompilerParams(needs_layout_passes=False),
        scratch_types=[
            pltpu.VMEM((_GSLICE,), jnp.float32),
            pltpu.VMEM((_HALF,), jnp.int32),
            pltpu.VMEM((_HALF,), jnp.int32),
            pltpu.VMEM((_HALF,), jnp.int32),
            pltpu.VMEM((_OUT_W,), jnp.float32),
        ],
    )
    def k(proj_hbm, p0_hbm, p1_hbm, dist_hbm, out_hbm,
          projv, p0v, p1v, dv, outv):
        wid = lax.axis_index("s") * _NC + lax.axis_index("c")
        iota = lax.iota(jnp.int32, 16)
        iota8 = iota * NUM_HEADS

        @pl.when(wid < (N_GRAPH - 1) * 2)
        def _compute():
            i = wid // 2
            half = wid % 2
            g = i + 1
            poff = pl.multiple_of(half * _HALF, 8)
            pltpu.sync_copy(
                proj_hbm.at[pl.ds(pl.multiple_of(i * _GSLICE, 8), _GSLICE)],
                projv)
            pltpu.sync_copy(p0_hbm.at[g, pl.ds(poff, _HALF)], p0v)
            pltpu.sync_copy(p1_hbm.at[g, pl.ds(poff, _HALF)], p1v)
            pltpu.sync_copy(dist_hbm.at[g, pl.ds(poff, _HALF)], dv)

            def body(b, _):
                base = b * 16
                s0 = p0v[pl.ds(base, 16)] * _PROJ_W
                s1 = p1v[pl.ds(base, 16)] * _PROJ_W + NUM_HEADS
                dvec = dv[pl.ds(base, 16)]
                rvec = jnp.where(dvec >= 2, jnp.float32(0.5), jnp.float32(1.0))
                ob = iota8 + base * NUM_HEADS
                for h in range(NUM_HEADS):
                    av = plsc.load_gather(projv, [s0 + h])
                    bv = plsc.load_gather(projv, [s1 + h])
                    plsc.store_scatter(outv, [ob + h], (av + bv) * rvec)
                return _

            lax.fori_loop(0, _HALF // 16, body, None)
            pltpu.sync_copy(
                outv,
                out_hbm.at[i, pl.ds(pl.multiple_of(half * _OUT_W, 8), _OUT_W)])

        @pl.when(wid >= _NW - 2)
        def _fill():
            half = wid - (_NW - 2)
            neg = jnp.full((16,), -1000.0, dtype=jnp.float32)

            def body(b, _):
                outv[pl.ds(b * 16, 16)] = neg
                return _

            lax.fori_loop(0, _OUT_W // 16, body, None)
            pltpu.sync_copy(
                outv,
                out_hbm.at[N_GRAPH - 1,
                           pl.ds(pl.multiple_of(half * _OUT_W, 8), _OUT_W)])

    return k(proj_flat, p0, p1, dist_flat)


def kernel(edge_feat, dist, path, emb_weight):
    proj = _proj_matmul(edge_feat, emb_weight)
    proj_flat = proj.reshape(-1)
    p0 = path[:, :, :, 0].reshape(N_GRAPH, _PAIRS)
    p1 = path[:, :, :, 1].reshape(N_GRAPH, _PAIRS)
    dist_flat = dist.reshape(N_GRAPH, _PAIRS)
    out = _sc_combine(proj_flat, p0, p1, dist_flat)
    return out.reshape(N_GRAPH, MAX_NODES, MAX_NODES, NUM_HEADS)
